# baseline (speedup 1.0000x reference)
import functools

import jax
import jax.numpy as jnp
from jax import lax
from jax.experimental import pallas as pl
from jax.experimental.pallas import tpu as pltpu
from jax.experimental.pallas import tpu_sc as plsc

_K = 16
_NJ = 8
_NCHUNK = 12544
_KPAD = _NJ * _NCHUNK
_CBL = 896
_NCB = _NCHUNK // _CBL
_QB = 128
_QBD = 64
_CAND = _K * _NJ
_BIG = 3e38
_PAD_VAL = 1e4


def _phase_a_body(xtest_ref, xtr_ref, m_ref):
    j = pl.program_id(1)
    xt = xtr_ref[...]
    n = jnp.sum(xt * xt, axis=1, keepdims=True)
    dots = lax.dot_general(
        xt, xtest_ref[...], (((1,), (1,)), ((), ())),
        preferred_element_type=jnp.float32)
    s = n - 2.0 * dots

    @pl.when(j == 0)
    def _first():
        m_ref[...] = s

    @pl.when(j > 0)
    def _rest():
        m_ref[...] = jnp.minimum(m_ref[...], s)


def _phase_b_body(m_ref, out_ref, cid_ref):
    def step(i, carry):
        iota = lax.broadcasted_iota(jnp.int32, (_NCHUNK, _QB), 0)
        m = jnp.min(m_ref[...], axis=0, keepdims=True)
        am = jnp.min(jnp.where(m_ref[...] == m, iota, _NCHUNK),
                     axis=0, keepdims=True)
        cid_ref[pl.ds(i, 1), :] = am
        m_ref[...] = jnp.where(iota == am, _BIG, m_ref[...])
        return carry

    lax.fori_loop(0, _K, step, 0)
    out_ref[...] = cid_ref[...]


def _phase_c_body(nq, xtr_hbm, idx_hbm, cid_hbm, y2_hbm, rows_out, yg_out,
                  idx_v, rows_v, cid_v, yr_v, sem):
    wid = lax.axis_index("s") * 2 + lax.axis_index("c")
    nw = 32
    b_per_w = (nq * _CAND) // nw
    nchunks = b_per_w // 128
    base = wid * b_per_w

    def chunk(c, carry):
        off = base + c * 128
        pltpu.sync_copy(idx_hbm.at[pl.ds(off, 128)], idx_v)
        pltpu.async_copy(xtr_hbm.at[idx_v], rows_v, sem).wait()
        pltpu.sync_copy(rows_v, rows_out.at[pl.ds(off, 128)])
        return carry

    lax.fori_loop(0, nchunks, chunk, 0)

    cb_per_w = (nq * _K) // nw
    cchunks = cb_per_w // 128
    cbase = wid * cb_per_w

    def ychunk(c, carry):
        off = cbase + c * 128
        pltpu.sync_copy(cid_hbm.at[pl.ds(off, 128)], cid_v)
        pltpu.async_copy(y2_hbm.at[cid_v], yr_v, sem).wait()
        pltpu.sync_copy(yr_v, yg_out.at[pl.ds(off, 128)])
        return carry

    lax.fori_loop(0, cchunks, ychunk, 0)


def _phase_d_body(xtest_ref, rows_ref, yg_ref, idx_ref, out_ref):
    n = jnp.sum(rows_ref[...] * rows_ref[...], axis=2)
    q = xtest_ref[...]
    nq = jnp.sum(q * q, axis=1, keepdims=True)
    dots = jnp.sum(rows_ref[...] * q[:, None, :], axis=2)
    s = jnp.sqrt(jnp.maximum(nq + (n - 2.0 * dots), 0.0))
    y = yg_ref[...]
    ridx = idx_ref[...]

    def step(i, carry):
        s, ysum = carry
        m = jnp.min(s, axis=1, keepdims=True)
        am = jnp.min(jnp.where(s == m, ridx, jnp.int32(2 ** 30)),
                     axis=1, keepdims=True)
        onehot = ridx == am
        ysum = ysum + jnp.sum(jnp.where(onehot, y, 0.0), axis=1,
                              keepdims=True)
        return jnp.where(onehot, _BIG, s), ysum

    _, ysum = lax.fori_loop(
        0, _K, step, (s, jnp.zeros((_QBD, 1), jnp.float32)))
    out_ref[...] = ysum / float(_K)


def _knn_half(x_pad, y2, x_test, d, interpret=False):
    qn = x_test.shape[0]

    m = pl.pallas_call(
        _phase_a_body,
        grid=(_NCB, _NJ),
        in_specs=[
            pl.BlockSpec((qn, d), lambda cb, j: (0, 0)),
            pl.BlockSpec((_CBL, d), lambda cb, j: (j * _NCB + cb, 0)),
        ],
        out_specs=pl.BlockSpec((_CBL, qn), lambda cb, j: (cb, 0)),
        out_shape=jax.ShapeDtypeStruct((_NCHUNK, qn), jnp.float32),
        compiler_params=pltpu.CompilerParams(
            dimension_semantics=("arbitrary", "arbitrary")),
        interpret=interpret,
    )(x_test, x_pad)

    cid_t = pl.pallas_call(
        _phase_b_body,
        grid=(qn // _QB,),
        in_specs=[pl.BlockSpec((_NCHUNK, _QB), lambda qb: (0, qb))],
        out_specs=pl.BlockSpec((_K, _QB), lambda qb: (0, qb)),
        out_shape=jax.ShapeDtypeStruct((_K, qn), jnp.int32),
        scratch_shapes=[pltpu.VMEM((_K, _QB), jnp.int32)],
        compiler_params=pltpu.CompilerParams(
            dimension_semantics=("arbitrary",)),
        interpret=interpret,
    )(m)
    cid = cid_t.T

    idx_flat = (cid[:, :, None]
                + jnp.arange(_NJ, dtype=jnp.int32)[None, None, :] * _NCHUNK
                ).reshape(-1)
    cid_flat = cid.reshape(-1)
    nb = qn * _CAND
    mesh = plsc.VectorSubcoreMesh(core_axis_name="c", subcore_axis_name="s")
    c_kernel = pl.kernel(
        functools.partial(_phase_c_body, qn),
        out_type=[
            jax.ShapeDtypeStruct((nb, d), jnp.float32),
            jax.ShapeDtypeStruct((qn * _K, d), jnp.float32),
        ],
        mesh=mesh,
        scratch_types=[
            pltpu.VMEM((128,), jnp.int32),
            pltpu.VMEM((128, d), jnp.float32),
            pltpu.VMEM((128,), jnp.int32),
            pltpu.VMEM((128, d), jnp.float32),
            pltpu.SemaphoreType.DMA,
        ],
        interpret=interpret,
    )
    rows, yg = c_kernel(x_pad, idx_flat, cid_flat, y2)

    rows3 = rows.reshape(qn, _CAND, d)
    yg2 = yg[:, :_NJ].reshape(qn, _CAND)
    idx3 = idx_flat.reshape(qn, _CAND)
    out = pl.pallas_call(
        _phase_d_body,
        grid=(qn // _QBD,),
        in_specs=[
            pl.BlockSpec((_QBD, d), lambda b: (b, 0)),
            pl.BlockSpec((_QBD, _CAND, d), lambda b: (b, 0, 0)),
            pl.BlockSpec((_QBD, _CAND), lambda b: (b, 0)),
            pl.BlockSpec((_QBD, _CAND), lambda b: (b, 0)),
        ],
        out_specs=pl.BlockSpec((_QBD, 1), lambda b: (b, 0)),
        out_shape=jax.ShapeDtypeStruct((qn, 1), jnp.float32),
        compiler_params=pltpu.CompilerParams(
            dimension_semantics=("arbitrary",)),
        interpret=interpret,
    )(x_test, rows3, yg2, idx3)
    return out[:, 0]


def _knn_pallas(x_train, x_test, y_train, interpret=False):
    ktot, d = x_train.shape
    qn = x_test.shape[0]
    x_pad = jnp.concatenate(
        [x_train,
         jnp.full((_KPAD - ktot, d), _PAD_VAL, jnp.float32)], axis=0)
    y_pad = jnp.concatenate(
        [y_train, jnp.zeros((_KPAD - ktot,), jnp.float32)], axis=0)
    y2 = jnp.pad(y_pad.reshape(_NJ, _NCHUNK).T, ((0, 0), (0, d - _NJ)))

    h = qn // 2
    out0 = _knn_half(x_pad, y2, x_test[:h], d, interpret)
    out1 = _knn_half(x_pad, y2, x_test[h:], d, interpret)
    return jnp.concatenate([out0, out1])


def kernel(X_train, X_test, y_train):
    return _knn_pallas(X_train, X_test, y_train)


# speedup vs baseline: 1.1085x; 1.1085x over previous
import functools

import jax
import jax.numpy as jnp
from jax import lax
from jax.experimental import pallas as pl
from jax.experimental.pallas import tpu as pltpu
from jax.experimental.pallas import tpu_sc as plsc

_K = 16
_NJ = 8
_NCHUNK = 12544
_KPAD = _NJ * _NCHUNK
_CBL = 896
_NCB = _NCHUNK // _CBL
_QB = 128
_QBD = 128
_CAND = _K * _NJ
_BIG = 3e38
_PAD_VAL = 1e4


def _phase_a_body(nq_ref, ntr_ref, xtest_ref, xtr_ref, m_ref):
    j = pl.program_id(1)
    xt = xtr_ref[...]
    dots = lax.dot_general(
        xt, xtest_ref[...], (((1,), (1,)), ((), ())),
        preferred_element_type=jnp.float32)
    s = (ntr_ref[...] + nq_ref[...]) - 2.0 * dots

    @pl.when(j == 0)
    def _first():
        m_ref[...] = s

    @pl.when(j > 0)
    def _rest():
        m_ref[...] = jnp.minimum(m_ref[...], s)


def _phase_b_body(m_ref, out_ref, cid_ref):
    def step(i, carry):
        iota = lax.broadcasted_iota(jnp.int32, (_NCHUNK, _QB), 0)
        m = jnp.min(m_ref[...], axis=0, keepdims=True)
        am = jnp.min(jnp.where(m_ref[...] == m, iota, _NCHUNK),
                     axis=0, keepdims=True)
        cid_ref[pl.ds(i, 1), :] = am
        m_ref[...] = jnp.where(iota == am, _BIG, m_ref[...])
        return carry

    lax.fori_loop(0, _K, step, 0)
    out_ref[...] = cid_ref[...]


def _phase_c_body(nq, xtr_hbm, idx_hbm, cid_hbm, y2_hbm, rows_out, yg_out,
                  idx_v, rows_v, cid_v, yr_v, sem):
    wid = lax.axis_index("s") * 2 + lax.axis_index("c")
    nw = 32
    b_per_w = (nq * _CAND) // nw
    nchunks = b_per_w // 128
    base = wid * b_per_w

    def chunk(c, carry):
        off = base + c * 128
        pltpu.sync_copy(idx_hbm.at[pl.ds(off, 128)], idx_v)
        pltpu.async_copy(xtr_hbm.at[idx_v], rows_v, sem).wait()
        pltpu.sync_copy(rows_v, rows_out.at[pl.ds(off, 128)])
        return carry

    lax.fori_loop(0, nchunks, chunk, 0)

    cb_per_w = (nq * _K) // nw
    cchunks = cb_per_w // 128
    cbase = wid * cb_per_w

    def ychunk(c, carry):
        off = cbase + c * 128
        pltpu.sync_copy(cid_hbm.at[pl.ds(off, 128)], cid_v)
        pltpu.async_copy(y2_hbm.at[cid_v], yr_v, sem).wait()
        pltpu.sync_copy(yr_v, yg_out.at[pl.ds(off, 128)])
        return carry

    lax.fori_loop(0, cchunks, ychunk, 0)


def _phase_d_body(xtest_ref, rows_ref, ygT_ref, ngT_ref, idxT_ref, nq_ref,
                  out_ref):
    dots = lax.dot_general(
        rows_ref[...], xtest_ref[...], (((1,), (1,)), ((), ())),
        preferred_element_type=jnp.float32)
    s3 = dots.reshape(_QBD, _CAND, _QBD)
    i0 = lax.broadcasted_iota(jnp.int32, (_QBD, _CAND, _QBD), 0)
    i2 = lax.broadcasted_iota(jnp.int32, (_QBD, _CAND, _QBD), 2)
    dsel = jnp.sum(jnp.where(i0 == i2, s3, 0.0), axis=0)
    sT = jnp.sqrt(jnp.maximum(
        (nq_ref[...] + ngT_ref[...]) - 2.0 * dsel, 0.0))
    yT = ygT_ref[...]
    ridx = idxT_ref[...]

    def step(i, carry):
        s, ysum = carry
        m = jnp.min(s, axis=0, keepdims=True)
        am = jnp.min(jnp.where(s == m, ridx, jnp.int32(2 ** 30)),
                     axis=0, keepdims=True)
        onehot = ridx == am
        ysum = ysum + jnp.sum(jnp.where(onehot, yT, 0.0), axis=0,
                              keepdims=True)
        return jnp.where(onehot, _BIG, s), ysum

    _, ysum = lax.fori_loop(
        0, _K, step, (sT, jnp.zeros((1, _QBD), jnp.float32)))
    out_ref[...] = ysum / float(_K)


def _knn_half(x_pad, y2, ntr2, nq2, x_test, d, interpret=False):
    qn = x_test.shape[0]

    m = pl.pallas_call(
        _phase_a_body,
        grid=(_NCB, _NJ),
        in_specs=[
            pl.BlockSpec((1, qn), lambda cb, j: (0, 0)),
            pl.BlockSpec((_CBL, 1), lambda cb, j: (j * _NCB + cb, 0)),
            pl.BlockSpec((qn, d), lambda cb, j: (0, 0)),
            pl.BlockSpec((_CBL, d), lambda cb, j: (j * _NCB + cb, 0)),
        ],
        out_specs=pl.BlockSpec((_CBL, qn), lambda cb, j: (cb, 0)),
        out_shape=jax.ShapeDtypeStruct((_NCHUNK, qn), jnp.float32),
        compiler_params=pltpu.CompilerParams(
            dimension_semantics=("arbitrary", "arbitrary")),
        interpret=interpret,
    )(nq2, ntr2, x_test, x_pad)

    cid_t = pl.pallas_call(
        _phase_b_body,
        grid=(qn // _QB,),
        in_specs=[pl.BlockSpec((_NCHUNK, _QB), lambda qb: (0, qb))],
        out_specs=pl.BlockSpec((_K, _QB), lambda qb: (0, qb)),
        out_shape=jax.ShapeDtypeStruct((_K, qn), jnp.int32),
        scratch_shapes=[pltpu.VMEM((_K, _QB), jnp.int32)],
        compiler_params=pltpu.CompilerParams(
            dimension_semantics=("arbitrary",)),
        interpret=interpret,
    )(m)
    cid = cid_t.T

    idx_flat = (cid[:, :, None]
                + jnp.arange(_NJ, dtype=jnp.int32)[None, None, :] * _NCHUNK
                ).reshape(-1)
    cid_flat = cid.reshape(-1)
    nb = qn * _CAND
    mesh = plsc.VectorSubcoreMesh(core_axis_name="c", subcore_axis_name="s")
    c_kernel = pl.kernel(
        functools.partial(_phase_c_body, qn),
        out_type=[
            jax.ShapeDtypeStruct((nb, d), jnp.float32),
            jax.ShapeDtypeStruct((qn * _K, d), jnp.float32),
        ],
        mesh=mesh,
        scratch_types=[
            pltpu.VMEM((128,), jnp.int32),
            pltpu.VMEM((128, d), jnp.float32),
            pltpu.VMEM((128,), jnp.int32),
            pltpu.VMEM((128, d), jnp.float32),
            pltpu.SemaphoreType.DMA,
        ],
        interpret=interpret,
    )
    rows, yg = c_kernel(x_pad, idx_flat, cid_flat, y2)

    ygT = yg[:, :_NJ].reshape(qn, _CAND).T
    ngT = yg[:, _NJ:2 * _NJ].reshape(qn, _CAND).T
    idxT = idx_flat.reshape(qn, _CAND).T
    out = pl.pallas_call(
        _phase_d_body,
        grid=(qn // _QBD,),
        in_specs=[
            pl.BlockSpec((_QBD, d), lambda b: (b, 0)),
            pl.BlockSpec((_QBD * _CAND, d), lambda b: (b, 0)),
            pl.BlockSpec((_CAND, _QBD), lambda b: (0, b)),
            pl.BlockSpec((_CAND, _QBD), lambda b: (0, b)),
            pl.BlockSpec((_CAND, _QBD), lambda b: (0, b)),
            pl.BlockSpec((1, _QBD), lambda b: (0, b)),
        ],
        out_specs=pl.BlockSpec((1, _QBD), lambda b: (0, b)),
        out_shape=jax.ShapeDtypeStruct((1, qn), jnp.float32),
        compiler_params=pltpu.CompilerParams(
            dimension_semantics=("arbitrary",)),
        interpret=interpret,
    )(x_test, rows, ygT, ngT, idxT, nq2)
    return out.reshape(-1)


def _knn_pallas(x_train, x_test, y_train, interpret=False):
    ktot, d = x_train.shape
    qn = x_test.shape[0]
    x_pad = jnp.concatenate(
        [x_train,
         jnp.full((_KPAD - ktot, d), _PAD_VAL, jnp.float32)], axis=0)
    y_pad = jnp.concatenate(
        [y_train, jnp.zeros((_KPAD - ktot,), jnp.float32)], axis=0)
    ntr = jnp.concatenate(
        [jnp.sum(x_train * x_train, axis=1),
         jnp.full((_KPAD - ktot,), 128.0 * _PAD_VAL * _PAD_VAL,
                  jnp.float32)], axis=0)
    nq = jnp.sum(x_test * x_test, axis=1)
    ntr2 = ntr[:, None]
    y2 = jnp.pad(
        jnp.concatenate([y_pad.reshape(_NJ, _NCHUNK).T,
                         ntr.reshape(_NJ, _NCHUNK).T], axis=1),
        ((0, 0), (0, d - 2 * _NJ)))

    return _knn_half(x_pad, y2, ntr2, nq[None, :], x_test, d, interpret)


def kernel(X_train, X_test, y_train):
    return _knn_pallas(X_train, X_test, y_train)
